# Initial kernel scaffold; baseline (speedup 1.0000x reference)
import jax
import jax.numpy as jnp
from jax.experimental import pallas as pl


def kernel(x, edge_index, batch, W1_rel, W1_root, b1, W2_rel, W2_root, b2, Wmu_rel, Wmu_root, bmu, Wls_rel, Wls_root, bls, Wp, bp, Wa, ba):
    raise NotImplementedError("write your pallas kernel here")


# speedup vs baseline: 5.7180x; 5.7180x over previous
import functools

import jax
import jax.numpy as jnp
from jax import lax
from jax.experimental import pallas as pl
from jax.experimental.pallas import tpu as pltpu
from jax.experimental.pallas import tpu_sc as plsc

N = 10000
E = 320000
D = 128
OC = 42
H1 = 16 * OC
H2 = 8 * OC
NG = 16
NA = 8
NP = 21

NCORE = 2
NSUB = 16
CH = 128
NCHK = 157
EPT = NCHK * CH
E_PAD = NSUB * EPT
TBL = 10016
RPT = TBL // NSUB

R = 1000
G = N // R


def _make_sc_agg(dcol):
  mesh = plsc.VectorSubcoreMesh(
      core_axis_name="c", subcore_axis_name="s",
      num_cores=NCORE, num_subcores=NSUB)

  @functools.partial(
      pl.kernel,
      out_type=jax.ShapeDtypeStruct((NCORE, TBL, dcol), jnp.float32),
      mesh=mesh,
      compiler_params=pltpu.CompilerParams(use_tc_tiling_on_sc=False),
      scratch_types=[
          pltpu.VMEM((EPT,), jnp.int32),
          pltpu.VMEM((NCHK, CH), jnp.int32),
          pltpu.VMEM((CH, dcol), jnp.float32),
          pltpu.VMEM_SHARED((TBL, dcol), jnp.float32),
      ],
  )
  def sc_agg(tbl_hbm, src_hbm, dst_hbm, zeros_hbm, out_hbm,
             src_v, dst_v, rows_v, shared):
    cid = lax.axis_index("c")
    sid = lax.axis_index("s")
    row0 = sid * RPT
    pltpu.sync_copy(zeros_hbm.at[pl.ds(row0, RPT)], shared.at[pl.ds(row0, RPT)])
    pltpu.sync_copy(src_hbm.at[cid * NSUB + sid], src_v)
    pltpu.sync_copy(dst_hbm.at[sid], dst_v)
    plsc.subcore_barrier()

    def step(c, carry):
      pltpu.sync_copy(tbl_hbm.at[src_v.at[pl.ds(c * CH, CH)]], rows_v)
      pltpu.sync_copy(rows_v, shared.at[dst_v.at[c]], add=True)
      return carry

    lax.fori_loop(0, NCHK, step, 0)
    plsc.subcore_barrier()
    pltpu.sync_copy(shared.at[pl.ds(row0, RPT)],
                    out_hbm.at[cid, pl.ds(row0, RPT)])

  return sc_agg


_sc_agg_88 = _make_sc_agg(88)
_sc_agg_48 = _make_sc_agg(48)


def _tc1_body(x_r, al_r, ah_r, cnt_r, w1rel_r, w1root_r, b1_r,
              w2rel_r, w2root_r, b2_r, t2a_r, t2b_r, r1_r):
  inv = 1.0 / jnp.maximum(cnt_r[...], 1.0)
  mean = jnp.concatenate([al_r[...], ah_r[...]], axis=1) * inv
  h1 = jnp.maximum(
      jnp.dot(mean, w1rel_r[...], preferred_element_type=jnp.float32)
      + jnp.dot(x_r[...], w1root_r[...], preferred_element_type=jnp.float32)
      + b1_r[...], 0.0)
  t1 = jnp.dot(h1, w2rel_r[...], preferred_element_type=jnp.float32)
  q = H2 // 4
  zpad = jnp.zeros((R, 4), jnp.float32)
  t2a_r[0] = jnp.concatenate([t1[:, 0 * q:1 * q], zpad], axis=1)
  t2a_r[1] = jnp.concatenate([t1[:, 2 * q:3 * q], zpad], axis=1)
  t2b_r[0] = jnp.concatenate([t1[:, 1 * q:2 * q], zpad], axis=1)
  t2b_r[1] = jnp.concatenate([t1[:, 3 * q:4 * q], zpad], axis=1)
  r1_r[...] = (jnp.dot(h1, w2root_r[...], preferred_element_type=jnp.float32)
               + b2_r[...])


def _tc2_body(a0_r, a1_r, a2_r, a3_r, cnt_r, r1_r, wmur_r, wlsr_r, wmuo_r,
              wlso_r, bmu_r, bls_r, t3_r, r2mu_r, r2ls_r):
  inv = 1.0 / jnp.maximum(cnt_r[...], 1.0)
  mean1 = jnp.concatenate(
      [a0_r[...], a1_r[...], a2_r[...], a3_r[...]], axis=1) * inv
  h2 = jnp.maximum(mean1 + r1_r[...], 0.0)
  t2mu = jnp.dot(h2, wmur_r[...], preferred_element_type=jnp.float32)
  t2ls = jnp.dot(h2, wlsr_r[...], preferred_element_type=jnp.float32)
  zpad = jnp.zeros((R, 6), jnp.float32)
  t3_r[0] = jnp.concatenate([t2mu, zpad], axis=1)
  t3_r[1] = jnp.concatenate([t2ls, zpad], axis=1)
  r2mu_r[...] = (jnp.dot(h2, wmuo_r[...], preferred_element_type=jnp.float32)
                 + bmu_r[...])
  r2ls_r[...] = (jnp.dot(h2, wlso_r[...], preferred_element_type=jnp.float32)
                 + bls_r[...])


def _tc3_body(amu_r, als_r, cnt_r, r2mu_r, r2ls_r, eps_r, batch_r,
              wp_r, bp_r, wa_r, ba_r, pz_r, z_r, gext_r, az_r):
  i = pl.program_id(0)
  inv = 1.0 / jnp.maximum(cnt_r[...], 1.0)
  mu = amu_r[...] * inv + r2mu_r[...]
  ls = jnp.minimum(als_r[...] * inv + r2ls_r[...], 10.0)
  z = mu + eps_r[...] * jnp.exp(ls)
  z_r[...] = z
  logits = jnp.maximum(
      jnp.dot(z, wp_r[...], preferred_element_type=jnp.float32) + bp_r[...],
      0.0)
  m = jnp.max(logits, axis=1, keepdims=True)
  pz_r[...] = (logits - m) - jnp.log(
      jnp.sum(jnp.exp(logits - m), axis=1, keepdims=True))
  lanes = lax.broadcasted_iota(jnp.int32, (R, NG), 1)
  oh = (batch_r[...] == lanes).astype(jnp.float32)
  zc = jnp.concatenate([z, jnp.ones((R, 1), jnp.float32)], axis=1)
  contrib = lax.dot_general(oh, zc, (((0,), (0,)), ((), ())),
                            preferred_element_type=jnp.float32)

  @pl.when(i == 0)
  def _():
    gext_r[...] = contrib

  @pl.when(i > 0)
  def _():
    gext_r[...] += contrib

  @pl.when(i == G - 1)
  def _():
    ge = gext_r[...]
    pooled = ge[:, :OC] / jnp.maximum(ge[:, OC:OC + 1], 1.0)
    al = jnp.dot(pooled, wa_r[...], preferred_element_type=jnp.float32) + ba_r[...]
    m2 = jnp.max(al, axis=1, keepdims=True)
    az_r[...] = (al - m2) - jnp.log(
        jnp.sum(jnp.exp(al - m2), axis=1, keepdims=True))


def _full(shape):
  return pl.BlockSpec(shape, lambda i: (0,) * len(shape))


def _rows(cols):
  return pl.BlockSpec((R, cols), lambda i: (i, 0))


def kernel(x, edge_index, batch, W1_rel, W1_root, b1, W2_rel, W2_root, b2,
           Wmu_rel, Wmu_root, bmu, Wls_rel, Wls_root, bls, Wp, bp, Wa, ba):
  src = edge_index[0]
  dst = edge_index[1]
  pad = E_PAD - E
  srcp = jnp.concatenate([src, jnp.zeros((pad,), jnp.int32)])
  dstp = jnp.concatenate([dst, jnp.full((pad,), N, jnp.int32)])
  src_t = srcp.reshape(NSUB, EPT)
  src2 = jnp.concatenate([src_t, src_t + N], axis=0)
  dst3 = dstp.reshape(NSUB, NCHK, CH)

  ones = jnp.ones((N, 1), jnp.float32)
  zc23 = jnp.zeros((N, 23), jnp.float32)
  ta = jnp.concatenate([
      jnp.concatenate([x[:, :64], ones, zc23], axis=1),
      jnp.concatenate([x[:, 64:], ones, zc23], axis=1)], axis=0)
  zeros88 = jnp.zeros((TBL, 88), jnp.float32)
  agg_a = _sc_agg_88(ta, src2, dst3, zeros88)
  aggl = agg_a[0, :N, :64]
  aggh = agg_a[1, :N, :64]
  cnt = agg_a[0, :N, 64:65]

  t2a, t2b, r1 = pl.pallas_call(
      _tc1_body,
      grid=(G,),
      in_specs=[_rows(D), _rows(64), _rows(64), _rows(1),
                _full((D, H1)), _full((D, H1)), _full((1, H1)),
                _full((H1, H2)), _full((H1, H2)), _full((1, H2))],
      out_specs=[pl.BlockSpec((2, R, 88), lambda i: (0, i, 0)),
                 pl.BlockSpec((2, R, 88), lambda i: (0, i, 0)),
                 _rows(H2)],
      out_shape=[jax.ShapeDtypeStruct((2, N, 88), jnp.float32),
                 jax.ShapeDtypeStruct((2, N, 88), jnp.float32),
                 jax.ShapeDtypeStruct((N, H2), jnp.float32)],
  )(x, aggl, aggh, cnt, W1_rel, W1_root, b1.reshape(1, H1),
    W2_rel, W2_root, b2.reshape(1, H2))

  q = H2 // 4
  agg_b1 = _sc_agg_88(t2a.reshape(2 * N, 88), src2, dst3, zeros88)
  agg_b2 = _sc_agg_88(t2b.reshape(2 * N, 88), src2, dst3, zeros88)
  a1q0 = agg_b1[0, :N, :q]
  a1q2 = agg_b1[1, :N, :q]
  a1q1 = agg_b2[0, :N, :q]
  a1q3 = agg_b2[1, :N, :q]

  t3_tab, r2mu, r2ls = pl.pallas_call(
      _tc2_body,
      grid=(G,),
      in_specs=[_rows(q), _rows(q), _rows(q), _rows(q), _rows(1), _rows(H2),
                _full((H2, OC)), _full((H2, OC)),
                _full((H2, OC)), _full((H2, OC)),
                _full((1, OC)), _full((1, OC))],
      out_specs=[pl.BlockSpec((2, R, 48), lambda i: (0, i, 0)),
                 _rows(OC), _rows(OC)],
      out_shape=[jax.ShapeDtypeStruct((2, N, 48), jnp.float32),
                 jax.ShapeDtypeStruct((N, OC), jnp.float32),
                 jax.ShapeDtypeStruct((N, OC), jnp.float32)],
  )(a1q0, a1q1, a1q2, a1q3, cnt, r1, Wmu_rel, Wls_rel, Wmu_root, Wls_root,
    bmu.reshape(1, OC), bls.reshape(1, OC))

  agg_c = _sc_agg_48(t3_tab.reshape(2 * N, 48), src2, dst3,
                     jnp.zeros((TBL, 48), jnp.float32))
  a2mu = agg_c[0, :N, :OC]
  a2ls = agg_c[1, :N, :OC]

  eps = jax.random.normal(jax.random.key(42), (N, OC), dtype=jnp.float32)
  p_z, z, _, a_z = pl.pallas_call(
      _tc3_body,
      grid=(G,),
      in_specs=[_rows(OC), _rows(OC), _rows(1), _rows(OC), _rows(OC),
                _rows(OC), _rows(1),
                _full((OC, NP)), _full((1, NP)),
                _full((OC, NA)), _full((1, NA))],
      out_specs=[_rows(NP), _rows(OC),
                 pl.BlockSpec((NG, OC + 1), lambda i: (0, 0)),
                 pl.BlockSpec((NG, NA), lambda i: (0, 0))],
      out_shape=[jax.ShapeDtypeStruct((N, NP), jnp.float32),
                 jax.ShapeDtypeStruct((N, OC), jnp.float32),
                 jax.ShapeDtypeStruct((NG, OC + 1), jnp.float32),
                 jax.ShapeDtypeStruct((NG, NA), jnp.float32)],
  )(a2mu, a2ls, cnt, r2mu, r2ls, eps, batch.reshape(N, 1),
    Wp, bp.reshape(1, NP), Wa, ba.reshape(1, NA))

  return (p_z, a_z, z)


# speedup vs baseline: 8.1032x; 1.4171x over previous
import functools

import jax
import jax.numpy as jnp
from jax import lax
from jax.experimental import pallas as pl
from jax.experimental.pallas import tpu as pltpu
from jax.experimental.pallas import tpu_sc as plsc

N = 10000
E = 320000
D = 128
OC = 42
H1 = 16 * OC
H2 = 8 * OC
NG = 16
NA = 8
NP = 21

NCORE = 2
NSUB = 16
CH = 128
NCHK = 157
EPT = NCHK * CH
E_PAD = NSUB * EPT
TBL = 10016
RPT = TBL // NSUB

NBUF = 2

R = 1000
G = N // R


def _make_sc_agg(dcol):
  mesh = plsc.VectorSubcoreMesh(
      core_axis_name="c", subcore_axis_name="s",
      num_cores=NCORE, num_subcores=NSUB)

  @functools.partial(
      pl.kernel,
      out_type=jax.ShapeDtypeStruct((NCORE, TBL, dcol), jnp.float32),
      mesh=mesh,
      compiler_params=pltpu.CompilerParams(use_tc_tiling_on_sc=False),
      scratch_types=[
          pltpu.VMEM((EPT,), jnp.int32),
          pltpu.VMEM((NCHK, CH), jnp.int32),
          pltpu.VMEM((NBUF, CH, dcol), jnp.float32),
          pltpu.VMEM_SHARED((TBL, dcol), jnp.float32),
      ] + [pltpu.SemaphoreType.DMA] * NBUF,
  )
  def sc_agg(tbl_hbm, src_hbm, dst_hbm, zeros_hbm, out_hbm,
             src_v, dst_v, rows_v, shared, *gsem):
    cid = lax.axis_index("c")
    sid = lax.axis_index("s")
    row0 = sid * RPT
    pltpu.sync_copy(zeros_hbm.at[pl.ds(row0, RPT)], shared.at[pl.ds(row0, RPT)])
    pltpu.sync_copy(src_hbm.at[cid * NSUB + sid], src_v)
    pltpu.sync_copy(dst_hbm.at[sid], dst_v)
    plsc.subcore_barrier()

    def g_desc(c, b):
      return pltpu.make_async_copy(
          tbl_hbm.at[src_v.at[pl.ds(c * CH, CH)]], rows_v.at[b], gsem[b])

    for b in range(NBUF):
      g_desc(b, b).start()

    def group(g, carry):
      for b in range(NBUF):
        c = g * NBUF + b
        g_desc(c, b).wait()
        pltpu.sync_copy(rows_v.at[b], shared.at[dst_v.at[c]], add=True)
        nxt = c + NBUF

        @pl.when(nxt < NCHK)
        def _():
          g_desc(nxt, b).start()

      return carry

    ngrp = NCHK // NBUF
    lax.fori_loop(0, ngrp, group, 0)
    for c in range(ngrp * NBUF, NCHK):
      b = c % NBUF
      g_desc(c, b).wait()
      pltpu.sync_copy(rows_v.at[b], shared.at[dst_v.at[c]], add=True)
    plsc.subcore_barrier()
    pltpu.sync_copy(shared.at[pl.ds(row0, RPT)],
                    out_hbm.at[cid, pl.ds(row0, RPT)])

  return sc_agg


_sc_agg_88 = _make_sc_agg(88)
_sc_agg_48 = _make_sc_agg(48)


def _tc1_body(x_r, al_r, ah_r, cnt_r, w1rel_r, w1root_r, b1_r,
              w2rel_r, w2root_r, b2_r, t2a_r, t2b_r, r1_r):
  inv = 1.0 / jnp.maximum(cnt_r[...], 1.0)
  mean = jnp.concatenate([al_r[...], ah_r[...]], axis=1) * inv
  h1 = jnp.maximum(
      jnp.dot(mean, w1rel_r[...], preferred_element_type=jnp.float32)
      + jnp.dot(x_r[...], w1root_r[...], preferred_element_type=jnp.float32)
      + b1_r[...], 0.0)
  t1 = jnp.dot(h1, w2rel_r[...], preferred_element_type=jnp.float32)
  q = H2 // 4
  zpad = jnp.zeros((R, 4), jnp.float32)
  t2a_r[0] = jnp.concatenate([t1[:, 0 * q:1 * q], zpad], axis=1)
  t2a_r[1] = jnp.concatenate([t1[:, 2 * q:3 * q], zpad], axis=1)
  t2b_r[0] = jnp.concatenate([t1[:, 1 * q:2 * q], zpad], axis=1)
  t2b_r[1] = jnp.concatenate([t1[:, 3 * q:4 * q], zpad], axis=1)
  r1_r[...] = (jnp.dot(h1, w2root_r[...], preferred_element_type=jnp.float32)
               + b2_r[...])


def _tc2_body(a0_r, a1_r, a2_r, a3_r, cnt_r, r1_r, wmur_r, wlsr_r, wmuo_r,
              wlso_r, bmu_r, bls_r, t3_r, r2mu_r, r2ls_r):
  inv = 1.0 / jnp.maximum(cnt_r[...], 1.0)
  mean1 = jnp.concatenate(
      [a0_r[...], a1_r[...], a2_r[...], a3_r[...]], axis=1) * inv
  h2 = jnp.maximum(mean1 + r1_r[...], 0.0)
  t2mu = jnp.dot(h2, wmur_r[...], preferred_element_type=jnp.float32)
  t2ls = jnp.dot(h2, wlsr_r[...], preferred_element_type=jnp.float32)
  zpad = jnp.zeros((R, 6), jnp.float32)
  t3_r[0] = jnp.concatenate([t2mu, zpad], axis=1)
  t3_r[1] = jnp.concatenate([t2ls, zpad], axis=1)
  r2mu_r[...] = (jnp.dot(h2, wmuo_r[...], preferred_element_type=jnp.float32)
                 + bmu_r[...])
  r2ls_r[...] = (jnp.dot(h2, wlso_r[...], preferred_element_type=jnp.float32)
                 + bls_r[...])


def _tc3_body(amu_r, als_r, cnt_r, r2mu_r, r2ls_r, eps_r, batch_r,
              wp_r, bp_r, wa_r, ba_r, pz_r, z_r, gext_r, az_r):
  i = pl.program_id(0)
  inv = 1.0 / jnp.maximum(cnt_r[...], 1.0)
  mu = amu_r[...] * inv + r2mu_r[...]
  ls = jnp.minimum(als_r[...] * inv + r2ls_r[...], 10.0)
  z = mu + eps_r[...] * jnp.exp(ls)
  z_r[...] = z
  logits = jnp.maximum(
      jnp.dot(z, wp_r[...], preferred_element_type=jnp.float32) + bp_r[...],
      0.0)
  m = jnp.max(logits, axis=1, keepdims=True)
  pz_r[...] = (logits - m) - jnp.log(
      jnp.sum(jnp.exp(logits - m), axis=1, keepdims=True))
  lanes = lax.broadcasted_iota(jnp.int32, (R, NG), 1)
  oh = (batch_r[...] == lanes).astype(jnp.float32)
  zc = jnp.concatenate([z, jnp.ones((R, 1), jnp.float32)], axis=1)
  contrib = lax.dot_general(oh, zc, (((0,), (0,)), ((), ())),
                            preferred_element_type=jnp.float32)

  @pl.when(i == 0)
  def _():
    gext_r[...] = contrib

  @pl.when(i > 0)
  def _():
    gext_r[...] += contrib

  @pl.when(i == G - 1)
  def _():
    ge = gext_r[...]
    pooled = ge[:, :OC] / jnp.maximum(ge[:, OC:OC + 1], 1.0)
    al = jnp.dot(pooled, wa_r[...], preferred_element_type=jnp.float32) + ba_r[...]
    m2 = jnp.max(al, axis=1, keepdims=True)
    az_r[...] = (al - m2) - jnp.log(
        jnp.sum(jnp.exp(al - m2), axis=1, keepdims=True))


def _full(shape):
  return pl.BlockSpec(shape, lambda i: (0,) * len(shape))


def _rows(cols):
  return pl.BlockSpec((R, cols), lambda i: (i, 0))


def kernel(x, edge_index, batch, W1_rel, W1_root, b1, W2_rel, W2_root, b2,
           Wmu_rel, Wmu_root, bmu, Wls_rel, Wls_root, bls, Wp, bp, Wa, ba):
  src = edge_index[0]
  dst = edge_index[1]
  pad = E_PAD - E
  srcp = jnp.concatenate([src, jnp.zeros((pad,), jnp.int32)])
  dstp = jnp.concatenate([dst, jnp.full((pad,), N, jnp.int32)])
  src_t = srcp.reshape(NSUB, EPT)
  src2 = jnp.concatenate([src_t, src_t + N], axis=0)
  dst3 = dstp.reshape(NSUB, NCHK, CH)

  ones = jnp.ones((N, 1), jnp.float32)
  zc23 = jnp.zeros((N, 23), jnp.float32)
  ta = jnp.concatenate([
      jnp.concatenate([x[:, :64], ones, zc23], axis=1),
      jnp.concatenate([x[:, 64:], ones, zc23], axis=1)], axis=0)
  zeros88 = jnp.zeros((TBL, 88), jnp.float32)
  agg_a = _sc_agg_88(ta, src2, dst3, zeros88)
  aggl = agg_a[0, :N, :64]
  aggh = agg_a[1, :N, :64]
  cnt = agg_a[0, :N, 64:65]

  t2a, t2b, r1 = pl.pallas_call(
      _tc1_body,
      grid=(G,),
      in_specs=[_rows(D), _rows(64), _rows(64), _rows(1),
                _full((D, H1)), _full((D, H1)), _full((1, H1)),
                _full((H1, H2)), _full((H1, H2)), _full((1, H2))],
      out_specs=[pl.BlockSpec((2, R, 88), lambda i: (0, i, 0)),
                 pl.BlockSpec((2, R, 88), lambda i: (0, i, 0)),
                 _rows(H2)],
      out_shape=[jax.ShapeDtypeStruct((2, N, 88), jnp.float32),
                 jax.ShapeDtypeStruct((2, N, 88), jnp.float32),
                 jax.ShapeDtypeStruct((N, H2), jnp.float32)],
  )(x, aggl, aggh, cnt, W1_rel, W1_root, b1.reshape(1, H1),
    W2_rel, W2_root, b2.reshape(1, H2))

  q = H2 // 4
  agg_b1 = _sc_agg_88(t2a.reshape(2 * N, 88), src2, dst3, zeros88)
  zeros88_dep = zeros88 + agg_b1[0, 0, 0] * 0.0
  agg_b2 = _sc_agg_88(t2b.reshape(2 * N, 88), src2, dst3, zeros88_dep)
  a1q0 = agg_b1[0, :N, :q]
  a1q2 = agg_b1[1, :N, :q]
  a1q1 = agg_b2[0, :N, :q]
  a1q3 = agg_b2[1, :N, :q]

  t3_tab, r2mu, r2ls = pl.pallas_call(
      _tc2_body,
      grid=(G,),
      in_specs=[_rows(q), _rows(q), _rows(q), _rows(q), _rows(1), _rows(H2),
                _full((H2, OC)), _full((H2, OC)),
                _full((H2, OC)), _full((H2, OC)),
                _full((1, OC)), _full((1, OC))],
      out_specs=[pl.BlockSpec((2, R, 48), lambda i: (0, i, 0)),
                 _rows(OC), _rows(OC)],
      out_shape=[jax.ShapeDtypeStruct((2, N, 48), jnp.float32),
                 jax.ShapeDtypeStruct((N, OC), jnp.float32),
                 jax.ShapeDtypeStruct((N, OC), jnp.float32)],
  )(a1q0, a1q1, a1q2, a1q3, cnt, r1, Wmu_rel, Wls_rel, Wmu_root, Wls_root,
    bmu.reshape(1, OC), bls.reshape(1, OC))

  agg_c = _sc_agg_48(t3_tab.reshape(2 * N, 48), src2, dst3,
                     jnp.zeros((TBL, 48), jnp.float32))
  a2mu = agg_c[0, :N, :OC]
  a2ls = agg_c[1, :N, :OC]

  eps = jax.random.normal(jax.random.key(42), (N, OC), dtype=jnp.float32)
  p_z, z, _, a_z = pl.pallas_call(
      _tc3_body,
      grid=(G,),
      in_specs=[_rows(OC), _rows(OC), _rows(1), _rows(OC), _rows(OC),
                _rows(OC), _rows(1),
                _full((OC, NP)), _full((1, NP)),
                _full((OC, NA)), _full((1, NA))],
      out_specs=[_rows(NP), _rows(OC),
                 pl.BlockSpec((NG, OC + 1), lambda i: (0, 0)),
                 pl.BlockSpec((NG, NA), lambda i: (0, 0))],
      out_shape=[jax.ShapeDtypeStruct((N, NP), jnp.float32),
                 jax.ShapeDtypeStruct((N, OC), jnp.float32),
                 jax.ShapeDtypeStruct((NG, OC + 1), jnp.float32),
                 jax.ShapeDtypeStruct((NG, NA), jnp.float32)],
  )(a2mu, a2ls, cnt, r2mu, r2ls, eps, batch.reshape(N, 1),
    Wp, bp.reshape(1, NP), Wa, ba.reshape(1, NA))

  return (p_z, a_z, z)
